# baseline (speedup 1.0000x reference)
import jax
import jax.numpy as jnp
from jax.experimental import pallas as pl
from jax.experimental.pallas import tpu as pltpu

_TB = 512
_NBUF = 4
_NSPLIT = 4
_ROWS = _TB // _NSPLIT


def _body(sel_hbm, items_ref, out_ref, buf, sems):
    n_chunks = out_ref.shape[0] // _TB
    items = items_ref[...]

    def copies(chunk, slot):
        for p in range(_NSPLIT):
            yield pltpu.make_async_copy(
                sel_hbm.at[pl.ds(chunk * _TB + p * _ROWS, _ROWS), :],
                buf.at[slot, pl.ds(p * _ROWS, _ROWS), :],
                sems.at[slot, p],
            )

    def start_copy(chunk, slot):
        for c in copies(chunk, slot):
            c.start()

    for k in range(_NBUF):
        start_copy(k, k)

    def step(i, _):
        slot = jax.lax.rem(i, _NBUF)
        for c in copies(i, slot):
            c.wait()
        s = buf[slot]
        m = jnp.max(s, axis=-1, keepdims=True)
        e = jnp.exp(s - m)
        z = jnp.sum(e, axis=-1, keepdims=True)
        acc = jnp.dot(e, items, preferred_element_type=jnp.float32)
        out_ref[pl.ds(i * _TB, _TB), :] = acc / z

        @pl.when(i + _NBUF < n_chunks)
        def _():
            start_copy(i + _NBUF, slot)

        return 0

    jax.lax.fori_loop(0, n_chunks, step, 0)


def kernel(selections, items):
    batch, n_items = selections.shape
    _, n_samples = items.shape
    return pl.pallas_call(
        _body,
        in_specs=[
            pl.BlockSpec(memory_space=pltpu.MemorySpace.HBM),
            pl.BlockSpec(memory_space=pltpu.MemorySpace.VMEM),
        ],
        out_specs=pl.BlockSpec(memory_space=pltpu.MemorySpace.VMEM),
        out_shape=jax.ShapeDtypeStruct((batch, n_samples), jnp.float32),
        scratch_shapes=[
            pltpu.VMEM((_NBUF, _TB, n_items), jnp.float32),
            pltpu.SemaphoreType.DMA((_NBUF, _NSPLIT)),
        ],
    )(selections, items)


# speedup vs baseline: 1.0487x; 1.0487x over previous
import jax
import jax.numpy as jnp
from jax.experimental import pallas as pl
from jax.experimental.pallas import tpu as pltpu

_TB = 512
_NBUF = 4
_NSPLIT = 4
_ROWS = _TB // _NSPLIT


def _body(sel_hbm, items_ref, out_ref, buf, sems):
    n_chunks = out_ref.shape[0] // _TB
    items = items_ref[...].astype(jnp.bfloat16)

    def copies(chunk, slot):
        for p in range(_NSPLIT):
            yield pltpu.make_async_copy(
                sel_hbm.at[pl.ds(chunk * _TB + p * _ROWS, _ROWS), :],
                buf.at[slot, pl.ds(p * _ROWS, _ROWS), :],
                sems.at[slot, p],
            )

    def start_copy(chunk, slot):
        for c in copies(chunk, slot):
            c.start()

    for k in range(_NBUF):
        start_copy(k, k)

    def step(i, _):
        slot = jax.lax.rem(i, _NBUF)
        for c in copies(i, slot):
            c.wait()
        s = buf[slot]
        m = jnp.max(s, axis=-1, keepdims=True)
        e = jnp.exp(s - m)
        z = jnp.sum(e, axis=-1, keepdims=True)
        acc = jnp.dot(e.astype(jnp.bfloat16), items, preferred_element_type=jnp.float32)
        out_ref[pl.ds(i * _TB, _TB), :] = acc / z

        @pl.when(i + _NBUF < n_chunks)
        def _():
            start_copy(i + _NBUF, slot)

        return 0

    jax.lax.fori_loop(0, n_chunks, step, 0)


def kernel(selections, items):
    batch, n_items = selections.shape
    _, n_samples = items.shape
    return pl.pallas_call(
        _body,
        in_specs=[
            pl.BlockSpec(memory_space=pltpu.MemorySpace.HBM),
            pl.BlockSpec(memory_space=pltpu.MemorySpace.VMEM),
        ],
        out_specs=pl.BlockSpec(memory_space=pltpu.MemorySpace.VMEM),
        out_shape=jax.ShapeDtypeStruct((batch, n_samples), jnp.float32),
        scratch_shapes=[
            pltpu.VMEM((_NBUF, _TB, n_items), jnp.float32),
            pltpu.SemaphoreType.DMA((_NBUF, _NSPLIT)),
        ],
    )(selections, items)
